# Initial kernel scaffold; baseline (speedup 1.0000x reference)
import jax
import jax.numpy as jnp
from jax.experimental import pallas as pl


def kernel(x, batch, W):
    raise NotImplementedError("write your pallas kernel here")


# speedup vs baseline: 2.2161x; 2.2161x over previous
import functools

import jax
import jax.numpy as jnp
from jax import lax
from jax.experimental import pallas as pl
from jax.experimental.pallas import tpu as pltpu
from jax.experimental.pallas import tpu_sc as plsc

_N = 100000
_D = 128
_S = 64
_NC = 2
_NS = 16
_NW = _NC * _NS
_CHUNK = 400
_NCHUNKS = _N // _CHUNK
_Q = 4
_QROWS = _CHUNK // _Q
_KMAX_EXTRA = _NCHUNKS % _NW
_KFULL = _NCHUNKS // _NW


def _sc_body(x_hbm, b_hbm, out_hbm, xbuf, bbuf, zbuf, shared):
    c = lax.axis_index("c")
    s = lax.axis_index("s")
    wid = c * _NS + s
    zv = jnp.zeros((16,), jnp.float32)

    @pl.when(s == 0)
    def _zero():
        def zrow(i, carry):
            for j8 in range(_D // 16):
                zbuf[i, pl.ds(j8 * 16, 16)] = zv
            return carry
        lax.fori_loop(0, _S, zrow, 0)
        pltpu.sync_copy(zbuf, shared)

    plsc.subcore_barrier()

    nk = jnp.where(wid < _KMAX_EXTRA, _KFULL + 1, _KFULL)

    def chunk_body(k, carry):
        ch = k * _NW + wid
        pltpu.sync_copy(x_hbm.at[pl.ds(ch * _CHUNK, _CHUNK)], xbuf)
        pltpu.sync_copy(b_hbm.at[ch], bbuf)
        for q in range(_Q):
            pltpu.sync_copy(
                xbuf.at[pl.ds(q * _QROWS, _QROWS)],
                shared.at[bbuf.at[q]],
                add=True,
            )
        return carry

    lax.fori_loop(0, nk, chunk_body, 0)

    plsc.subcore_barrier()

    @pl.when(s == 0)
    def _publish():
        pltpu.sync_copy(shared, out_hbm.at[c])


_sc_segment_sum = functools.partial(
    pl.kernel,
    out_type=jax.ShapeDtypeStruct((_NC, _S, _D), jnp.float32),
    mesh=plsc.VectorSubcoreMesh(
        core_axis_name="c", subcore_axis_name="s",
        num_cores=_NC, num_subcores=_NS,
    ),
    scratch_types=[
        pltpu.VMEM((_CHUNK, _D), jnp.float32),
        pltpu.VMEM((_Q, _QROWS), jnp.int32),
        pltpu.VMEM((_S, _D), jnp.float32),
        pltpu.VMEM_SHARED((_S, _D), jnp.float32),
    ],
)(_sc_body)


def _readout_body(f_ref, w_ref, o_ref):
    w_row = w_ref[...].reshape(1, _D)
    folded = f_ref[0] + f_ref[1]
    o_ref[...] = jnp.sum(folded * w_row, axis=1)


_readout = pl.pallas_call(
    _readout_body,
    out_shape=jax.ShapeDtypeStruct((_S,), jnp.float32),
)


def kernel(x, batch, W):
    b3 = batch.astype(jnp.int32).reshape(_NCHUNKS, _Q, _QROWS)
    feat = _sc_segment_sum(x, b3)
    return _readout(feat, W.reshape(_D))


# speedup vs baseline: 2.3833x; 1.0755x over previous
import functools

import jax
import jax.numpy as jnp
from jax import lax
from jax.experimental import pallas as pl
from jax.experimental.pallas import tpu as pltpu
from jax.experimental.pallas import tpu_sc as plsc

_N = 100000
_D = 128
_S = 64
_NC = 2
_NS = 16
_NW = _NC * _NS
_CHUNK = 80
_NCHUNKS = _N // _CHUNK
_KEXTRA = _NCHUNKS % _NW
_KFULL = _NCHUNKS // _NW


def _sc_body(x_hbm, b_hbm, out_hbm,
             xb0, xb1, bb0, bb1, zbuf, shared, sx0, sb0, sx1, sb1):
    c = lax.axis_index("c")
    s = lax.axis_index("s")
    wid = c * _NS + s
    zv = jnp.zeros((16,), jnp.float32)

    @pl.when(s == 0)
    def _zero_shared():
        def zrow(i, carry):
            for j8 in range(_D // 16):
                zbuf[i, pl.ds(j8 * 16, 16)] = zv
            return carry

        lax.fori_loop(0, _S, zrow, 0)
        pltpu.sync_copy(zbuf, shared)

    plsc.subcore_barrier()

    nk = jnp.where(wid < _KEXTRA, _KFULL + 1, _KFULL)

    pltpu.async_copy(x_hbm.at[pl.ds(wid * _CHUNK, _CHUNK)], xb0, sx0)
    pltpu.async_copy(b_hbm.at[wid], bb0, sb0)

    def turn(k, xb_cur, bb_cur, sx_cur, sb_cur, xb_nxt, bb_nxt, sx_nxt, sb_nxt):
        @pl.when(k < nk)
        def _():
            @pl.when(k + 1 < nk)
            def _issue_next():
                ch = (k + 1) * _NW + wid
                pltpu.async_copy(x_hbm.at[pl.ds(ch * _CHUNK, _CHUNK)],
                                 xb_nxt, sx_nxt)
                pltpu.async_copy(b_hbm.at[ch], bb_nxt, sb_nxt)

            pltpu.make_async_copy(
                x_hbm.at[pl.ds(0, _CHUNK)], xb_cur, sx_cur).wait()
            pltpu.make_async_copy(b_hbm.at[0], bb_cur, sb_cur).wait()
            pltpu.sync_copy(xb_cur, shared.at[bb_cur.at[0]], add=True)

    def body(g, carry):
        turn(2 * g, xb0, bb0, sx0, sb0, xb1, bb1, sx1, sb1)
        turn(2 * g + 1, xb1, bb1, sx1, sb1, xb0, bb0, sx0, sb0)
        return carry

    lax.fori_loop(0, (_KFULL + 2) // 2, body, 0)

    plsc.subcore_barrier()

    @pl.when(s == 0)
    def _publish():
        pltpu.sync_copy(shared, out_hbm.at[c])


_sc_segment_sum = functools.partial(
    pl.kernel,
    out_type=jax.ShapeDtypeStruct((_NC, _S, _D), jnp.float32),
    mesh=plsc.VectorSubcoreMesh(
        core_axis_name="c", subcore_axis_name="s",
        num_cores=_NC, num_subcores=_NS,
    ),
    scratch_types=[
        pltpu.VMEM((_CHUNK, _D), jnp.float32),
        pltpu.VMEM((_CHUNK, _D), jnp.float32),
        pltpu.VMEM((1, _CHUNK), jnp.int32),
        pltpu.VMEM((1, _CHUNK), jnp.int32),
        pltpu.VMEM((_S, _D), jnp.float32),
        pltpu.VMEM_SHARED((_S, _D), jnp.float32),
        pltpu.SemaphoreType.DMA,
        pltpu.SemaphoreType.DMA,
        pltpu.SemaphoreType.DMA,
        pltpu.SemaphoreType.DMA,
    ],
)(_sc_body)


def _readout_body(f_ref, w_ref, o_ref):
    w_row = w_ref[...].reshape(1, _D)
    folded = f_ref[0] + f_ref[1]
    o_ref[...] = jnp.sum(folded * w_row, axis=1)


_readout = pl.pallas_call(
    _readout_body,
    out_shape=jax.ShapeDtypeStruct((_S,), jnp.float32),
)


def kernel(x, batch, W):
    b2 = batch.astype(jnp.int32).reshape(_NCHUNKS, 1, _CHUNK)
    feat = _sc_segment_sum(x, b2)
    return _readout(feat, W.reshape(_D))
